# baseline (speedup 1.0000x reference)
import functools
import jax
import jax.numpy as jnp
from jax import lax
from jax.experimental import pallas as pl
from jax.experimental.pallas import tpu as pltpu
from jax.experimental.pallas import tpu_sc as plsc

EMB = 64
NC = 2
NS = 16
LANES = 16


@functools.partial(jax.jit, static_argnames=("batch",))
def _sc_gather(uidx3, iidx3, udiv3, idiv3, umod3, imod3,
               deep_user, deep_item, lu16, li16, batch):
    nw = NC * NS
    b_per_w = batch // nw
    chunk = 128
    n_chunks = b_per_w // chunk
    mesh = plsc.VectorSubcoreMesh(core_axis_name="c", subcore_axis_name="s")

    @functools.partial(
        pl.kernel,
        mesh=mesh,
        compiler_params=pltpu.CompilerParams(use_tc_tiling_on_sc=False,
                                             needs_layout_passes=False),
        out_type=[
            jax.ShapeDtypeStruct((batch, EMB), jnp.float32),
            jax.ShapeDtypeStruct((batch, EMB), jnp.float32),
            jax.ShapeDtypeStruct((batch,), jnp.float32),
        ],
        scratch_types=[
            pltpu.VMEM((n_chunks, chunk), jnp.int32),
            pltpu.VMEM((n_chunks, chunk), jnp.int32),
            pltpu.VMEM((n_chunks, chunk), jnp.int32),
            pltpu.VMEM((n_chunks, chunk), jnp.int32),
            pltpu.VMEM((n_chunks, chunk), jnp.int32),
            pltpu.VMEM((n_chunks, chunk), jnp.int32),
            pltpu.VMEM((b_per_w, EMB), jnp.float32),
            pltpu.VMEM((b_per_w, EMB), jnp.float32),
            pltpu.VMEM((b_per_w, LANES), jnp.float32),
            pltpu.VMEM((b_per_w, LANES), jnp.float32),
            pltpu.VMEM((b_per_w,), jnp.float32),
            pltpu.SemaphoreType.DMA,
        ],
    )
    def gather_kernel(uidx_hbm, iidx_hbm, udiv_hbm, idiv_hbm, umod_hbm,
                      imod_hbm, du_hbm, di_hbm, lu_hbm, li_hbm,
                      ue_hbm, ie_hbm, wide_hbm,
                      uidx_v, iidx_v, udiv_v, idiv_v, umod_v, imod_v,
                      urows_v, irows_v, wu_v, wi_v, wide_v, sem):
        wid = lax.axis_index("s") * NC + lax.axis_index("c")
        base = wid * b_per_w
        pltpu.sync_copy(uidx_hbm.at[wid], uidx_v)
        pltpu.sync_copy(iidx_hbm.at[wid], iidx_v)
        pltpu.sync_copy(udiv_hbm.at[wid], udiv_v)
        pltpu.sync_copy(idiv_hbm.at[wid], idiv_v)
        pltpu.sync_copy(umod_hbm.at[wid], umod_v)
        pltpu.sync_copy(imod_hbm.at[wid], imod_v)
        copies = []
        for j in range(n_chunks):
            dst = pl.ds(j * chunk, chunk)
            copies.append(pltpu.async_copy(
                du_hbm.at[uidx_v.at[j]], urows_v.at[dst], sem))
            copies.append(pltpu.async_copy(
                di_hbm.at[iidx_v.at[j]], irows_v.at[dst], sem))
            copies.append(pltpu.async_copy(
                lu_hbm.at[udiv_v.at[j]], wu_v.at[dst], sem))
            copies.append(pltpu.async_copy(
                li_hbm.at[idiv_v.at[j]], wi_v.at[dst], sem))
        for cp in copies:
            cp.wait()
        for g in range(b_per_w // LANES):
            gbase = g * LANES
            j = gbase // chunk
            off = gbase % chunk
            rows = lax.iota(jnp.int32, LANES) + gbase
            cols_u = umod_v[j, pl.ds(off, LANES)]
            cols_i = imod_v[j, pl.ds(off, LANES)]
            vu = plsc.load_gather(wu_v, [rows, cols_u])
            vi = plsc.load_gather(wi_v, [rows, cols_i])
            wide_v[pl.ds(gbase, LANES)] = vu + vi
        pltpu.sync_copy(urows_v, ue_hbm.at[pl.ds(base, b_per_w)])
        pltpu.sync_copy(irows_v, ie_hbm.at[pl.ds(base, b_per_w)])
        pltpu.sync_copy(wide_v, wide_hbm.at[pl.ds(base, b_per_w)])

    return gather_kernel(uidx3, iidx3, udiv3, idiv3, umod3, imod3,
                         deep_user, deep_item, lu16, li16)


def _mlp_body(ue, ie, wide, w1u, w1i, b1, w2, b2, out):
    h = jnp.dot(ue[...], w1u[...], preferred_element_type=jnp.float32)
    h = h + jnp.dot(ie[...], w1i[...], preferred_element_type=jnp.float32)
    h = jnp.maximum(h + b1[...], 0.0)
    deep = jnp.sum(h * w2[...], axis=1, keepdims=True)
    out[...] = deep + b2[...] + wide[...]


@jax.jit
def _tc_mlp(ue, ie, wide, w1u, w1i, b1, w2, b2):
    batch = ue.shape[0]
    blk = 2048
    grid = (batch // blk,)
    return pl.pallas_call(
        _mlp_body,
        grid=grid,
        in_specs=[
            pl.BlockSpec((blk, EMB), lambda i: (i, 0)),
            pl.BlockSpec((blk, EMB), lambda i: (i, 0)),
            pl.BlockSpec((blk, 1), lambda i: (i, 0)),
            pl.BlockSpec((EMB, EMB), lambda i: (0, 0)),
            pl.BlockSpec((EMB, EMB), lambda i: (0, 0)),
            pl.BlockSpec((1, EMB), lambda i: (0, 0)),
            pl.BlockSpec((1, EMB), lambda i: (0, 0)),
            pl.BlockSpec((1, 1), lambda i: (0, 0)),
        ],
        out_specs=pl.BlockSpec((blk, 1), lambda i: (i, 0)),
        out_shape=jax.ShapeDtypeStruct((batch, 1), jnp.float32),
    )(ue, ie, wide, w1u, w1i, b1, w2, b2)


def kernel(user_idx, item_idx, linear_user, linear_item, deep_user,
           deep_item, W1, b1, W2, b2):
    batch = user_idx.shape[0]
    nw = NC * NS
    chunk = 128
    n_chunks = batch // nw // chunk
    uidx = user_idx.astype(jnp.int32)
    iidx = item_idx.astype(jnp.int32)
    uidx3 = uidx.reshape(nw, n_chunks, chunk)
    iidx3 = iidx.reshape(nw, n_chunks, chunk)
    udiv3 = (uidx3 >> 4)
    idiv3 = (iidx3 >> 4)
    umod3 = (uidx3 & 15)
    imod3 = (iidx3 & 15)
    lu16 = linear_user.reshape(-1, LANES)
    li16 = linear_item.reshape(-1, LANES)
    ue, ie, wide = _sc_gather(uidx3, iidx3, udiv3, idiv3, umod3, imod3,
                              deep_user, deep_item, lu16, li16, batch)
    w1u = W1[:, :EMB].T
    w1i = W1[:, EMB:].T
    return _tc_mlp(ue, ie, wide.reshape(batch, 1), w1u, w1i,
                   b1.reshape(1, EMB), W2.reshape(1, EMB),
                   b2.reshape(1, 1))
